# baseline (speedup 1.0000x reference)
import functools

import jax
import jax.numpy as jnp
from jax import lax
from jax.experimental import pallas as pl
from jax.experimental.pallas import tpu as pltpu
from jax.experimental.pallas import tpu_sc as plsc

N_NODES = 10000
N_EDGES = 160000

NC = 2
NS = 16
NW = NC * NS

D = 8
Z = 10240
EPT = N_EDGES // NW
OCH = 1000
RPS = Z // NS


def _vmesh():
    return plsc.VectorSubcoreMesh(core_axis_name="c", subcore_axis_name="s")


def _deg_body(dst_hbm, ones_hbm, zeros_hbm, out_hbm, idx_v, ones_v, tab_sh, sem):
    cid = lax.axis_index("c")
    sid = lax.axis_index("s")
    wid = sid * NC + cid
    pltpu.sync_copy(dst_hbm.at[wid], idx_v)
    pltpu.sync_copy(ones_hbm, ones_v)
    pltpu.sync_copy(zeros_hbm.at[pl.ds(sid * RPS, RPS)],
                    tab_sh.at[pl.ds(sid * RPS, RPS)])
    plsc.subcore_barrier()
    for k in range(EPT // OCH):
        pltpu.sync_copy(ones_v, tab_sh.at[idx_v.at[k]], add=True)
    plsc.subcore_barrier()
    pltpu.sync_copy(tab_sh.at[pl.ds(sid * RPS, RPS)],
                    out_hbm.at[cid, pl.ds(sid * RPS, RPS)])


def _sc_degree(dst3, ones8, zeros8):
    return pl.kernel(
        _deg_body,
        out_type=jax.ShapeDtypeStruct((NC, Z, D), jnp.float32),
        mesh=_vmesh(),
        compiler_params=pltpu.CompilerParams(use_tc_tiling_on_sc=False),
        scratch_types=[
            pltpu.VMEM((EPT // OCH, OCH), jnp.int32),
            pltpu.VMEM((OCH, D), jnp.float32),
            pltpu.VMEM_SHARED((Z, D), jnp.float32),
            pltpu.SemaphoreType.DMA,
        ],
    )(dst3, ones8, zeros8)


def _newton_rsqrt(d):
    i = plsc.bitcast(d, jnp.int32)
    i = jnp.int32(0x5F3759DF) - lax.shift_right_logical(i, 1)
    g = plsc.bitcast(i, jnp.float32)
    for _ in range(3):
        g = g * (1.5 - 0.5 * d * g * g)
    return g


def _agg_body(src_hbm, dst_hbm, xw4_hbm, dp_hbm, zeros_hbm, out_hbm,
              src_v, dst_v, rows_v, xw_v, p0_v, p1_v, dis_v, ybuf_v, cbuf_v,
              id_v, ytab_sh, atab_sh, sem):
    cid = lax.axis_index("c")
    sid = lax.axis_index("s")
    wid = sid * NC + cid
    base = sid * RPS

    pltpu.sync_copy(src_hbm.at[wid], src_v)
    pltpu.sync_copy(dst_hbm.at[wid], dst_v)
    pltpu.sync_copy(xw4_hbm.at[pl.ds(base, RPS)], xw_v)
    pltpu.sync_copy(dp_hbm.at[0, pl.ds(base, RPS)], p0_v)
    pltpu.sync_copy(dp_hbm.at[1, pl.ds(base, RPS)], p1_v)
    pltpu.sync_copy(zeros_hbm.at[pl.ds(base, RPS)],
                    atab_sh.at[pl.ds(base, RPS)])

    lanes = lax.iota(jnp.int32, 16)
    colpat = lax.bitwise_and(lanes, 7)
    colmask4 = colpat < 4
    rowhalf = lanes // 8
    zcol = jnp.zeros((16,), jnp.int32)

    def build(k, carry):
        r0 = k * 16
        d = (plsc.load_gather(p0_v, [r0 + lanes, zcol])
             + plsc.load_gather(p1_v, [r0 + lanes, zcol]) + 1.0)
        dis = _newton_rsqrt(d)
        dis_v[pl.ds(r0, 16)] = dis
        id_v[pl.ds(r0, 16)] = base + r0 + lanes
        for j in range(8):
            noderel = r0 + 2 * j + rowhalf
            xwv = plsc.load_gather(xw_v, [noderel, colpat], mask=colmask4)
            disv = plsc.load_gather(dis_v, [noderel])
            plsc.store_scatter(ybuf_v, [noderel, colpat], xwv * disv)
        return carry

    lax.fori_loop(0, RPS // 16, build, 0)
    pltpu.sync_copy(ybuf_v, ytab_sh.at[pl.ds(base, RPS)])
    plsc.subcore_barrier()

    pltpu.async_copy(ytab_sh.at[src_v], rows_v, sem).wait()
    pltpu.sync_copy(rows_v, atab_sh.at[dst_v], add=True)

    @pl.when(cid == 0)
    def _selfterm():
        pltpu.sync_copy(ybuf_v, atab_sh.at[id_v], add=True)

    plsc.subcore_barrier()
    pltpu.sync_copy(atab_sh.at[pl.ds(base, RPS)], p0_v)

    def post(k, carry):
        r0 = k * 16
        for j in range(8):
            noderel = r0 + 2 * j + rowhalf
            av = plsc.load_gather(p0_v, [noderel, colpat])
            disv = plsc.load_gather(dis_v, [noderel])
            plsc.store_scatter(cbuf_v, [noderel, colpat], av * disv,
                               mask=colmask4)
        return carry

    lax.fori_loop(0, RPS // 16, post, 0)
    pltpu.sync_copy(cbuf_v, out_hbm.at[cid, pl.ds(base, RPS)])


def _sc_aggregate(src2, dst2, xw4, deg_parts, zeros8):
    return pl.kernel(
        _agg_body,
        out_type=jax.ShapeDtypeStruct((NC, Z, 4), jnp.float32),
        mesh=_vmesh(),
        compiler_params=pltpu.CompilerParams(use_tc_tiling_on_sc=False,
                                             needs_layout_passes=False),
        scratch_types=[
            pltpu.VMEM((EPT,), jnp.int32),
            pltpu.VMEM((EPT,), jnp.int32),
            pltpu.VMEM((EPT, D), jnp.float32),
            pltpu.VMEM((RPS, 4), jnp.float32),
            pltpu.VMEM((RPS, D), jnp.float32),
            pltpu.VMEM((RPS, D), jnp.float32),
            pltpu.VMEM((RPS,), jnp.float32),
            pltpu.VMEM((RPS, D), jnp.float32),
            pltpu.VMEM((RPS, 4), jnp.float32),
            pltpu.VMEM((RPS,), jnp.int32),
            pltpu.VMEM_SHARED((Z, D), jnp.float32),
            pltpu.VMEM_SHARED((Z, D), jnp.float32),
            pltpu.SemaphoreType.DMA,
        ],
    )(src2, dst2, xw4, deg_parts, zeros8)


def _mm_kernel(x_ref, w_ref, o_ref):
    blk = x_ref.shape[0]
    o_ref[...] = jnp.concatenate(
        [jnp.dot(x_ref[...], w_ref[...], preferred_element_type=jnp.float32),
         jnp.ones((blk, 1), jnp.float32)], axis=1)


def _tc_matmul(x, W_gcn):
    blk = 2048
    return pl.pallas_call(
        _mm_kernel,
        grid=(Z // blk,),
        in_specs=[pl.BlockSpec((blk, 256), lambda i: (i, 0)),
                  pl.BlockSpec((256, 3), lambda i: (0, 0))],
        out_specs=pl.BlockSpec((blk, 4), lambda i: (i, 0)),
        out_shape=jax.ShapeDtypeStruct((Z, 4), jnp.float32),
    )(x, W_gcn)


def _final_kernel(a0_ref, a1_ref, bg_ref, wo_ref, bo_ref, h_ref, z_ref):
    aggsum = a0_ref[:, 0:3] + a1_ref[:, 0:3]
    h = jnp.maximum(aggsum + bg_ref[...], 0.0)
    h_ref[...] = h
    z_ref[...] = (h[:, 0:1] * wo_ref[0:1, :] + h[:, 1:2] * wo_ref[1:2, :]
                  + h[:, 2:3] * wo_ref[2:3, :] + bo_ref[...])


def _tc_final(a0, a1, b_gcn, W_out, b_out):
    blk = 2000
    return pl.pallas_call(
        _final_kernel,
        grid=(N_NODES // blk,),
        in_specs=[pl.BlockSpec((blk, 4), lambda i: (i, 0)),
                  pl.BlockSpec((blk, 4), lambda i: (i, 0)),
                  pl.BlockSpec((1, 3), lambda i: (0, 0)),
                  pl.BlockSpec((3, 64), lambda i: (0, 0)),
                  pl.BlockSpec((1, 64), lambda i: (0, 0))],
        out_specs=[pl.BlockSpec((blk, 3), lambda i: (i, 0)),
                   pl.BlockSpec((blk, 64), lambda i: (i, 0))],
        out_shape=[jax.ShapeDtypeStruct((N_NODES, 3), jnp.float32),
                   jax.ShapeDtypeStruct((N_NODES, 64), jnp.float32)],
    )(a0, a1, b_gcn, W_out, b_out)


def kernel(x, edge_index, W_gcn, b_gcn, W_out, b_out):
    src2 = edge_index[0].astype(jnp.int32).reshape(NW, EPT)
    dst2 = edge_index[1].astype(jnp.int32).reshape(NW, EPT)
    dst3 = dst2.reshape(NW, EPT // OCH, OCH)
    zeros8 = jnp.zeros((Z, D), jnp.float32)
    ones8 = jnp.ones((OCH, D), jnp.float32)

    deg_parts = _sc_degree(dst3, ones8, zeros8)
    xw4 = _tc_matmul(x, W_gcn)
    agg = _sc_aggregate(src2, dst2, xw4, deg_parts, zeros8)

    h, z = _tc_final(agg[0, :N_NODES, :], agg[1, :N_NODES, :],
                     b_gcn.reshape(1, 3), W_out, b_out.reshape(1, 64))
    return (h, z)


# speedup vs baseline: 1.0100x; 1.0100x over previous
import functools

import jax
import jax.numpy as jnp
from jax import lax
from jax.experimental import pallas as pl
from jax.experimental.pallas import tpu as pltpu
from jax.experimental.pallas import tpu_sc as plsc

N_NODES = 10000
N_EDGES = 160000

NC = 2
NS = 16
NW = NC * NS

D = 8
Z = 10240
EPT = N_EDGES // NW
OCH = 1000
RPS = Z // NS


def _vmesh():
    return plsc.VectorSubcoreMesh(core_axis_name="c", subcore_axis_name="s")


def _deg_body(dst_hbm, ones_hbm, zeros_hbm, out_hbm, idx_v, ones_v, tab_sh, sem):
    cid = lax.axis_index("c")
    sid = lax.axis_index("s")
    wid = sid * NC + cid
    pltpu.sync_copy(dst_hbm.at[wid], idx_v)
    pltpu.sync_copy(ones_hbm, ones_v)
    pltpu.sync_copy(zeros_hbm.at[pl.ds(sid * RPS, RPS)],
                    tab_sh.at[pl.ds(sid * RPS, RPS)])
    plsc.subcore_barrier()
    for k in range(EPT // OCH):
        pltpu.sync_copy(ones_v, tab_sh.at[idx_v.at[k]], add=True)
    plsc.subcore_barrier()
    pltpu.sync_copy(tab_sh.at[pl.ds(sid * RPS, RPS)],
                    out_hbm.at[cid, pl.ds(sid * RPS, RPS)])


def _sc_degree(dst3, ones8, zeros8):
    return pl.kernel(
        _deg_body,
        out_type=jax.ShapeDtypeStruct((NC, Z, D), jnp.float32),
        mesh=_vmesh(),
        compiler_params=pltpu.CompilerParams(use_tc_tiling_on_sc=False),
        scratch_types=[
            pltpu.VMEM((EPT // OCH, OCH), jnp.int32),
            pltpu.VMEM((OCH, D), jnp.float32),
            pltpu.VMEM_SHARED((Z, D), jnp.float32),
            pltpu.SemaphoreType.DMA,
        ],
    )(dst3, ones8, zeros8)


def _newton_rsqrt(d):
    i = plsc.bitcast(d, jnp.int32)
    i = jnp.int32(0x5F3759DF) - lax.shift_right_logical(i, 1)
    g = plsc.bitcast(i, jnp.float32)
    for _ in range(3):
        g = g * (1.5 - 0.5 * d * g * g)
    return g


def _agg_body(src_hbm, dst_hbm, xw4_hbm, dp_hbm, zeros_hbm, out_hbm,
              src_v, dst_v, rows_v, xw_v, p0_v, p1_v, dis_v, ybuf_v, cbuf_v,
              id_v, ytab_sh, atab_sh, sem):
    cid = lax.axis_index("c")
    sid = lax.axis_index("s")
    wid = sid * NC + cid
    base = sid * RPS

    pltpu.sync_copy(src_hbm.at[wid], src_v)
    pltpu.sync_copy(dst_hbm.at[wid], dst_v)
    pltpu.sync_copy(xw4_hbm.at[pl.ds(base, RPS)], xw_v)
    pltpu.sync_copy(dp_hbm.at[0, pl.ds(base, RPS)], p0_v)
    pltpu.sync_copy(dp_hbm.at[1, pl.ds(base, RPS)], p1_v)
    pltpu.sync_copy(zeros_hbm.at[pl.ds(base, RPS)],
                    atab_sh.at[pl.ds(base, RPS)])

    lanes = lax.iota(jnp.int32, 16)
    colpat = lax.bitwise_and(lanes, 7)
    colmask4 = colpat < 4
    rowhalf = lanes // 8
    zcol = jnp.zeros((16,), jnp.int32)

    def build(k, carry):
        r0 = k * 16
        d = (plsc.load_gather(p0_v, [r0 + lanes, zcol])
             + plsc.load_gather(p1_v, [r0 + lanes, zcol]) + 1.0)
        dis = _newton_rsqrt(d)
        dis_v[pl.ds(r0, 16)] = dis
        id_v[pl.ds(r0, 16)] = base + r0 + lanes
        for j in range(8):
            noderel = r0 + 2 * j + rowhalf
            xwv = plsc.load_gather(xw_v, [noderel, colpat], mask=colmask4)
            disv = plsc.load_gather(dis_v, [noderel])
            plsc.store_scatter(ybuf_v, [noderel, colpat], xwv * disv)
        return carry

    lax.fori_loop(0, RPS // 16, build, 0)
    pltpu.sync_copy(ybuf_v, ytab_sh.at[pl.ds(base, RPS)])
    plsc.subcore_barrier()

    pltpu.async_copy(ytab_sh.at[src_v], rows_v, sem).wait()
    pltpu.sync_copy(rows_v, atab_sh.at[dst_v], add=True)

    @pl.when(cid == 0)
    def _selfterm():
        pltpu.sync_copy(ybuf_v, atab_sh.at[id_v], add=True)

    plsc.subcore_barrier()
    pltpu.sync_copy(atab_sh.at[pl.ds(base, RPS)], p0_v)

    def post(k, carry):
        r0 = k * 16
        for j in range(8):
            noderel = r0 + 2 * j + rowhalf
            av = plsc.load_gather(p0_v, [noderel, colpat])
            disv = plsc.load_gather(dis_v, [noderel])
            plsc.store_scatter(cbuf_v, [noderel, colpat], av * disv,
                               mask=colmask4)
        return carry

    lax.fori_loop(0, RPS // 16, post, 0)
    pltpu.sync_copy(cbuf_v, out_hbm.at[cid, pl.ds(base, RPS)])


def _sc_aggregate(src2, dst2, xw4, deg_parts, zeros8):
    return pl.kernel(
        _agg_body,
        out_type=jax.ShapeDtypeStruct((NC, Z, 4), jnp.float32),
        mesh=_vmesh(),
        compiler_params=pltpu.CompilerParams(use_tc_tiling_on_sc=False,
                                             needs_layout_passes=False),
        scratch_types=[
            pltpu.VMEM((EPT,), jnp.int32),
            pltpu.VMEM((EPT,), jnp.int32),
            pltpu.VMEM((EPT, D), jnp.float32),
            pltpu.VMEM((RPS, 4), jnp.float32),
            pltpu.VMEM((RPS, D), jnp.float32),
            pltpu.VMEM((RPS, D), jnp.float32),
            pltpu.VMEM((RPS,), jnp.float32),
            pltpu.VMEM((RPS, D), jnp.float32),
            pltpu.VMEM((RPS, 4), jnp.float32),
            pltpu.VMEM((RPS,), jnp.int32),
            pltpu.VMEM_SHARED((Z, D), jnp.float32),
            pltpu.VMEM_SHARED((Z, D), jnp.float32),
            pltpu.SemaphoreType.DMA,
        ],
    )(src2, dst2, xw4, deg_parts, zeros8)


def _mm_kernel(x_ref, w_ref, o_ref):
    blk = x_ref.shape[0]
    o_ref[...] = jnp.concatenate(
        [jnp.dot(x_ref[...], w_ref[...], preferred_element_type=jnp.float32),
         jnp.ones((blk, 1), jnp.float32)], axis=1)


def _tc_matmul(x, W_gcn):
    blk = 2048
    return pl.pallas_call(
        _mm_kernel,
        grid=(Z // blk,),
        in_specs=[pl.BlockSpec((blk, 256), lambda i: (i, 0)),
                  pl.BlockSpec((256, 3), lambda i: (0, 0))],
        out_specs=pl.BlockSpec((blk, 4), lambda i: (i, 0)),
        out_shape=jax.ShapeDtypeStruct((Z, 4), jnp.float32),
    )(x, W_gcn)


def _final_kernel(a_ref, bg_ref, wo_ref, bo_ref, h_ref, z_ref):
    aggsum = a_ref[0, :, 0:3] + a_ref[1, :, 0:3]
    h = jnp.maximum(aggsum + bg_ref[...], 0.0)
    h_ref[...] = h
    z_ref[...] = jnp.dot(h, wo_ref[...],
                         preferred_element_type=jnp.float32) + bo_ref[...]


def _tc_final(aggn, b_gcn, W_out, b_out):
    blk = 2000
    return pl.pallas_call(
        _final_kernel,
        grid=(N_NODES // blk,),
        in_specs=[pl.BlockSpec((2, blk, 4), lambda i: (0, i, 0)),
                  pl.BlockSpec((1, 3), lambda i: (0, 0)),
                  pl.BlockSpec((3, 64), lambda i: (0, 0)),
                  pl.BlockSpec((1, 64), lambda i: (0, 0))],
        out_specs=[pl.BlockSpec((blk, 3), lambda i: (i, 0)),
                   pl.BlockSpec((blk, 64), lambda i: (i, 0))],
        out_shape=[jax.ShapeDtypeStruct((N_NODES, 3), jnp.float32),
                   jax.ShapeDtypeStruct((N_NODES, 64), jnp.float32)],
    )(aggn, b_gcn, W_out, b_out)


def kernel(x, edge_index, W_gcn, b_gcn, W_out, b_out):
    src2 = edge_index[0].astype(jnp.int32).reshape(NW, EPT)
    dst2 = edge_index[1].astype(jnp.int32).reshape(NW, EPT)
    dst3 = dst2.reshape(NW, EPT // OCH, OCH)
    zeros8 = jnp.zeros((Z, D), jnp.float32)
    ones8 = jnp.ones((OCH, D), jnp.float32)

    deg_parts = _sc_degree(dst3, ones8, zeros8)
    xw4 = _tc_matmul(x, W_gcn)
    agg = _sc_aggregate(src2, dst2, xw4, deg_parts, zeros8)

    h, z = _tc_final(agg[:, :N_NODES, :],
                     b_gcn.reshape(1, 3), W_out, b_out.reshape(1, 64))
    return (h, z)
